# baseline (speedup 1.0000x reference)
import functools
import math

import jax
from jax import lax
import jax.numpy as jnp
from jax.experimental import pallas as pl
from jax.experimental.pallas import tpu as pltpu
from jax.experimental.pallas import tpu_sc as plsc

_H = 4
_K = 16
_INTERVAL = 3


def _proj_kernel(wq_ref, wk_ref, wv_ref, x_ref, q_ref, k_ref, v_ref):
    x = x_ref[...]
    q_ref[...] = jax.lax.dot(wq_ref[...], x, preferred_element_type=jnp.float32)
    k_ref[...] = jax.lax.dot(wk_ref[...], x, preferred_element_type=jnp.float32)
    v_ref[...] = jax.lax.dot(wv_ref[...], x, preferred_element_type=jnp.float32)


def _make_sc_knn(BT, J, N):
    nw = 32
    rows_per_w = (BT * N) // nw
    seg_per_bt = N // rows_per_w
    mesh = plsc.VectorSubcoreMesh(core_axis_name="c", subcore_axis_name="s")

    @functools.partial(
        pl.kernel,
        mesh=mesh,
        compiler_params=pltpu.CompilerParams(needs_layout_passes=False),
        out_type=jax.ShapeDtypeStruct((BT * N * 16,), jnp.float32),
        scratch_types=[
            pltpu.VMEM((3, J), jnp.float32),
            pltpu.VMEM((rows_per_w * 3 * 16,), jnp.float32),
            pltpu.VMEM((rows_per_w * 16,), jnp.float32),
        ],
    )
    def sc_knn(cand_hbm, pts_hbm, top_hbm, cnd_v, pts_v, top_v):
        wid = lax.axis_index("s") * 2 + lax.axis_index("c")
        bt = wid // seg_per_bt
        pltpu.sync_copy(cand_hbm.at[bt], cnd_v)
        pltpu.sync_copy(pts_hbm.at[wid], pts_v)
        inf16 = jnp.full((16,), jnp.inf, jnp.float32)

        def blk_body(blk, carry):
            base = blk * 16 * 48
            pxs = [pts_v[pl.ds(base + r * 48, 16)] for r in range(16)]
            pys = [pts_v[pl.ds(base + r * 48 + 16, 16)] for r in range(16)]
            pzs = [pts_v[pl.ds(base + r * 48 + 32, 16)] for r in range(16)]

            def ch_body(j, Ss):
                cx = cnd_v[0, pl.ds(j * 16, 16)]
                cy = cnd_v[1, pl.ds(j * 16, 16)]
                cz = cnd_v[2, pl.ds(j * 16, 16)]
                out = []
                for r in range(16):
                    dx = pxs[r] - cx
                    dy = pys[r] - cy
                    dz = pzs[r] - cz
                    d2 = dx * dx + dy * dy + dz * dz
                    dsrt, _ = plsc.sort_key_val(d2, d2, descending=True)
                    merged = jnp.minimum(Ss[r], dsrt)
                    msrt, _ = plsc.sort_key_val(merged, merged)
                    out.append(msrt)
                return tuple(out)

            Ss = lax.fori_loop(0, J // 16, ch_body, tuple([inf16] * 16))
            for r in range(16):
                top_v[pl.ds((blk * 16 + r) * 16, 16)] = Ss[r]
            return carry

        lax.fori_loop(0, rows_per_w // 16, blk_body, 0)
        pltpu.sync_copy(
            top_v, top_hbm.at[pl.ds(wid * rows_per_w * 16, rows_per_w * 16)])

    return sc_knn


def _attn_kernel(pos_ref, cand_ref, th_ref, q_ref, x_ref, vx_ref, kc_ref,
                 vc_ref, out_ref):
    R = pos_ref.shape[1]
    J = cand_ref.shape[2]
    pos = pos_ref[0]
    cnd = cand_ref[0]

    d2 = jnp.zeros((R, J), jnp.float32)
    for c in range(3):
        diff = pos[:, c][:, None] - cnd[c, :][None, :]
        d2 = d2 + diff * diff

    kf = float(_K)
    th = th_ref[0, 0, :][:, None]
    lt = d2 < th
    eqm = d2 == th
    mprev = jnp.sum(lt.astype(jnp.float32), axis=1, keepdims=True)
    ecur = jnp.sum(eqm.astype(jnp.float32), axis=1, keepdims=True)
    frac = jnp.where(ecur > 0.0,
                     jnp.maximum(kf - mprev, 0.0) / jnp.maximum(ecur, 1.0),
                     0.0)
    w = lt.astype(jnp.float32) + eqm.astype(jnp.float32) * frac

    x = x_ref[0]
    q = q_ref[0]
    vx = vx_ref[0]
    d = q.shape[1] // _H
    scale = 1.0 / math.sqrt(d)
    for h in range(_H):
        sl = slice(h * d, (h + 1) * d)
        qh = q[:, sl]
        kch = kc_ref[0, sl, :]
        vch = vc_ref[0, sl, :]
        en = (-scale) * jax.lax.dot(qh, kch, preferred_element_type=jnp.float32)
        mx = jnp.max(jnp.where(w > 0.0, en, -jnp.inf), axis=1, keepdims=True)
        p = w * jnp.exp(jnp.minimum(en - mx, 0.0))
        s = jnp.sum(p, axis=1, keepdims=True)
        av = jax.lax.dot_general(p, vch, (((1,), (1,)), ((), ())),
                                 preferred_element_type=jnp.float32)
        out_ref[0, :, sl] = x[:, sl] + vx[:, sl] - av / s


def _bn_mlp_kernel(xr_ref, g_ref, b_ref, w1_ref, w2_ref, out_ref):
    xr = xr_ref[0]
    mean = jnp.mean(xr, axis=1, keepdims=True)
    var = jnp.mean((xr - mean) * (xr - mean), axis=1, keepdims=True)
    g = g_ref[0][:, None]
    b = b_ref[0][:, None]
    xn = (xr - mean) / jnp.sqrt(var + 1e-5) * g + b
    h1 = jax.lax.dot(w1_ref[...], xn, preferred_element_type=jnp.float32)
    h1 = jnp.where(h1 > 0.0, h1, 0.2 * h1)
    out_ref[0] = jax.lax.dot(w2_ref[...], h1, preferred_element_type=jnp.float32)


def kernel(x_all, Wq, Wk, Wv, W1, W2, gamma1, beta1):
    B, C, T, N = x_all.shape
    BT = B * T
    J = _INTERVAL * N
    R = 256
    Cout = W2.shape[0]
    Cmid = W1.shape[0]

    x_flat = x_all.transpose(1, 0, 2, 3).reshape(C, BT * N)
    q_flat, kp_flat, vp_flat = pl.pallas_call(
        _proj_kernel,
        out_shape=[jax.ShapeDtypeStruct((C, BT * N), jnp.float32)] * 3,
    )(Wq, Wk, Wv, x_flat)

    def windows(arr):
        Ch = arr.shape[1]
        pad0 = arr[:, :, :1]
        padT = arr[:, :, T - 1:]
        ap = jnp.concatenate([pad0, arr, padT], axis=2)
        st = jnp.stack([ap[:, :, t:t + _INTERVAL].reshape(B, Ch, J)
                        for t in range(T)], axis=1)
        return st.reshape(BT, Ch, J)

    kp = kp_flat.reshape(C, B, T, N).transpose(1, 0, 2, 3)
    vp = vp_flat.reshape(C, B, T, N).transpose(1, 0, 2, 3)
    kc = windows(kp)
    vc = windows(vp)
    cand3 = windows(x_all[:, :3])

    SPLITS = 4
    BTs = BT // SPLITS

    def pts_for(xpos_bt):
        rows = (BTs * N) // 32
        nseg = N // rows
        p = (xpos_bt.reshape(BTs, 3, nseg, rows)
             .transpose(0, 2, 3, 1))
        p = jnp.broadcast_to(p[..., None], p.shape + (16,))
        return p.reshape(32, rows * 3 * 16)

    xpos = x_all[:, :3].transpose(0, 2, 1, 3).reshape(BT, 3, N)
    sc_knn = _make_sc_knn(BTs, J, N)

    pos_pts = x_all[:, :4].transpose(0, 2, 3, 1).reshape(BT, N, 4)
    x_in = x_all.transpose(0, 2, 3, 1).reshape(BT, N, C)
    q_in = q_flat.reshape(C, B, T, N).transpose(1, 2, 3, 0).reshape(BT, N, C)
    vx_in = vp_flat.reshape(C, B, T, N).transpose(1, 2, 3, 0).reshape(BT, N, C)

    thetas = []
    for sp in range(SPLITS):
        sl = slice(sp * BTs, (sp + 1) * BTs)
        top16 = sc_knn(cand3[sl], pts_for(xpos[sl]))
        thetas.append(top16.reshape(BTs * N, 16)[:, 15].reshape(BTs, 1, N))

    xr_parts = []
    for sp in range(SPLITS):
        sl = slice(sp * BTs, (sp + 1) * BTs)
        xr_parts.append(pl.pallas_call(
            _attn_kernel,
            grid=(BTs, N // R),
            in_specs=[
                pl.BlockSpec((1, R, 4), lambda bt, nt: (bt, nt, 0)),
                pl.BlockSpec((1, 3, J), lambda bt, nt: (bt, 0, 0)),
                pl.BlockSpec((1, 1, R), lambda bt, nt: (bt, 0, nt)),
                pl.BlockSpec((1, R, C), lambda bt, nt: (bt, nt, 0)),
                pl.BlockSpec((1, R, C), lambda bt, nt: (bt, nt, 0)),
                pl.BlockSpec((1, R, C), lambda bt, nt: (bt, nt, 0)),
                pl.BlockSpec((1, C, J), lambda bt, nt: (bt, 0, 0)),
                pl.BlockSpec((1, C, J), lambda bt, nt: (bt, 0, 0)),
            ],
            out_specs=pl.BlockSpec((1, R, C), lambda bt, nt: (bt, nt, 0)),
            out_shape=jax.ShapeDtypeStruct((BTs, N, C), jnp.float32),
        )(pos_pts[sl], cand3[sl], thetas[sp], q_in[sl], x_in[sl],
          vx_in[sl], kc[sl], vc[sl]))
    xr = jnp.concatenate(xr_parts, axis=0)

    xr_t = xr.reshape(B, T, N, C).transpose(1, 3, 0, 2).reshape(T, C, B * N)
    y = pl.pallas_call(
        _bn_mlp_kernel,
        grid=(T,),
        in_specs=[
            pl.BlockSpec((1, C, B * N), lambda t: (t, 0, 0)),
            pl.BlockSpec((1, C), lambda t: (0, 0)),
            pl.BlockSpec((1, C), lambda t: (0, 0)),
            pl.BlockSpec((Cmid, C), lambda t: (0, 0)),
            pl.BlockSpec((Cout, Cmid), lambda t: (0, 0)),
        ],
        out_specs=pl.BlockSpec((1, Cout, B * N), lambda t: (t, 0, 0)),
        out_shape=jax.ShapeDtypeStruct((T, Cout, B * N), jnp.float32),
    )(xr_t, gamma1.reshape(1, C), beta1.reshape(1, C), W1, W2)

    y = y.reshape(T, Cout, B, N).transpose(2, 1, 0, 3)
    return jnp.concatenate([x_all[:, :4], y], axis=1)


# speedup vs baseline: 1.2086x; 1.2086x over previous
import functools
import math

import jax
from jax import lax
import jax.numpy as jnp
from jax.experimental import pallas as pl
from jax.experimental.pallas import tpu as pltpu
from jax.experimental.pallas import tpu_sc as plsc

_H = 4
_K = 16
_INTERVAL = 3


def _proj_kernel(wq_ref, wk_ref, wv_ref, x_ref, q_ref, k_ref, v_ref):
    x = x_ref[...]
    q_ref[...] = jax.lax.dot(wq_ref[...], x, preferred_element_type=jnp.float32)
    k_ref[...] = jax.lax.dot(wk_ref[...], x, preferred_element_type=jnp.float32)
    v_ref[...] = jax.lax.dot(wv_ref[...], x, preferred_element_type=jnp.float32)


def _make_sc_knn(BT, J, N):
    nw = 32
    rows_per_w = (BT * N) // nw
    seg_per_bt = N // rows_per_w
    mesh = plsc.VectorSubcoreMesh(core_axis_name="c", subcore_axis_name="s")

    @functools.partial(
        pl.kernel,
        mesh=mesh,
        compiler_params=pltpu.CompilerParams(needs_layout_passes=False),
        out_type=jax.ShapeDtypeStruct((BT * N * 16,), jnp.float32),
        scratch_types=[
            pltpu.VMEM((3, J), jnp.float32),
            pltpu.VMEM((rows_per_w * 3 * 16,), jnp.float32),
            pltpu.VMEM((rows_per_w * 16,), jnp.float32),
        ],
    )
    def sc_knn(cand_hbm, pts_hbm, top_hbm, cnd_v, pts_v, top_v):
        wid = lax.axis_index("s") * 2 + lax.axis_index("c")
        bt = wid // seg_per_bt
        pltpu.sync_copy(cand_hbm.at[bt], cnd_v)
        pltpu.sync_copy(pts_hbm.at[wid], pts_v)
        inf16 = jnp.full((16,), jnp.inf, jnp.float32)

        def blk_body(blk, carry):
            base = blk * 16 * 48
            pxs = [pts_v[pl.ds(base + r * 48, 16)] for r in range(16)]
            pys = [pts_v[pl.ds(base + r * 48 + 16, 16)] for r in range(16)]
            pzs = [pts_v[pl.ds(base + r * 48 + 32, 16)] for r in range(16)]

            def ch_body(j, Ss):
                cx = cnd_v[0, pl.ds(j * 16, 16)]
                cy = cnd_v[1, pl.ds(j * 16, 16)]
                cz = cnd_v[2, pl.ds(j * 16, 16)]
                out = []
                for r in range(16):
                    dx = pxs[r] - cx
                    dy = pys[r] - cy
                    dz = pzs[r] - cz
                    d2 = dx * dx + dy * dy + dz * dz
                    dsrt, _ = plsc.sort_key_val(d2, d2, descending=True)
                    merged = jnp.minimum(Ss[r], dsrt)
                    msrt, _ = plsc.sort_key_val(merged, merged)
                    out.append(msrt)
                return tuple(out)

            Ss = lax.fori_loop(0, J // 16, ch_body, tuple([inf16] * 16))
            for r in range(16):
                top_v[pl.ds((blk * 16 + r) * 16, 16)] = Ss[r]
            return carry

        lax.fori_loop(0, rows_per_w // 16, blk_body, 0)
        pltpu.sync_copy(
            top_v, top_hbm.at[pl.ds(wid * rows_per_w * 16, rows_per_w * 16)])

    return sc_knn


def _attn_kernel(pos_ref, cand_ref, th_ref, q_ref, x_ref, vx_ref, kc_ref,
                 vc_ref, out_ref):
    R = pos_ref.shape[1]
    J = cand_ref.shape[2]
    pos = pos_ref[0]
    cnd = cand_ref[0]

    d2 = jnp.zeros((R, J), jnp.float32)
    for c in range(3):
        diff = pos[:, c][:, None] - cnd[c, :][None, :]
        d2 = d2 + diff * diff

    kf = float(_K)
    th = th_ref[0, 0, :][:, None]
    lt = d2 < th
    eqm = d2 == th
    mprev = jnp.sum(lt.astype(jnp.float32), axis=1, keepdims=True)
    ecur = jnp.sum(eqm.astype(jnp.float32), axis=1, keepdims=True)
    frac = jnp.where(ecur > 0.0,
                     jnp.maximum(kf - mprev, 0.0) / jnp.maximum(ecur, 1.0),
                     0.0)
    w = lt.astype(jnp.float32) + eqm.astype(jnp.float32) * frac

    x = x_ref[0]
    q = q_ref[0]
    vx = vx_ref[0]
    d = q.shape[1] // _H
    scale = 1.0 / math.sqrt(d)
    for h in range(_H):
        sl = slice(h * d, (h + 1) * d)
        qh = q[:, sl]
        kch = kc_ref[0, sl, :]
        vch = vc_ref[0, sl, :]
        en = (-scale) * jax.lax.dot(qh, kch, preferred_element_type=jnp.float32)
        mx = jnp.max(en, axis=1, keepdims=True)
        p = w * jnp.exp(en - mx)
        s = jnp.sum(p, axis=1, keepdims=True)
        av = jax.lax.dot_general(p, vch, (((1,), (1,)), ((), ())),
                                 preferred_element_type=jnp.float32)
        out_ref[0, :, sl] = x[:, sl] + vx[:, sl] - av / s


def _bn_mlp_kernel(xr_ref, g_ref, b_ref, w1_ref, w2_ref, out_ref):
    xr = xr_ref[0]
    mean = jnp.mean(xr, axis=1, keepdims=True)
    var = jnp.mean((xr - mean) * (xr - mean), axis=1, keepdims=True)
    g = g_ref[0][:, None]
    b = b_ref[0][:, None]
    xn = (xr - mean) / jnp.sqrt(var + 1e-5) * g + b
    h1 = jax.lax.dot(w1_ref[...], xn, preferred_element_type=jnp.float32)
    h1 = jnp.where(h1 > 0.0, h1, 0.2 * h1)
    out_ref[0] = jax.lax.dot(w2_ref[...], h1, preferred_element_type=jnp.float32)


def kernel(x_all, Wq, Wk, Wv, W1, W2, gamma1, beta1):
    B, C, T, N = x_all.shape
    BT = B * T
    J = _INTERVAL * N
    R = 256
    Cout = W2.shape[0]
    Cmid = W1.shape[0]

    x_flat = x_all.transpose(1, 0, 2, 3).reshape(C, BT * N)
    q_flat, kp_flat, vp_flat = pl.pallas_call(
        _proj_kernel,
        out_shape=[jax.ShapeDtypeStruct((C, BT * N), jnp.float32)] * 3,
    )(Wq, Wk, Wv, x_flat)

    def windows(arr):
        Ch = arr.shape[1]
        pad0 = arr[:, :, :1]
        padT = arr[:, :, T - 1:]
        ap = jnp.concatenate([pad0, arr, padT], axis=2)
        st = jnp.stack([ap[:, :, t:t + _INTERVAL].reshape(B, Ch, J)
                        for t in range(T)], axis=1)
        return st.reshape(BT, Ch, J)

    kp = kp_flat.reshape(C, B, T, N).transpose(1, 0, 2, 3)
    vp = vp_flat.reshape(C, B, T, N).transpose(1, 0, 2, 3)
    kc = windows(kp)
    vc = windows(vp)
    cand3 = windows(x_all[:, :3])

    SPLITS = 2
    BTs = BT // SPLITS

    def pts_for(xpos_bt):
        rows = (BTs * N) // 32
        nseg = N // rows
        p = (xpos_bt.reshape(BTs, 3, nseg, rows)
             .transpose(0, 2, 3, 1))
        p = jnp.broadcast_to(p[..., None], p.shape + (16,))
        return p.reshape(32, rows * 3 * 16)

    xpos = x_all[:, :3].transpose(0, 2, 1, 3).reshape(BT, 3, N)
    sc_knn = _make_sc_knn(BTs, J, N)

    pos_pts = x_all[:, :4].transpose(0, 2, 3, 1).reshape(BT, N, 4)
    x_in = x_all.transpose(0, 2, 3, 1).reshape(BT, N, C)
    q_in = q_flat.reshape(C, B, T, N).transpose(1, 2, 3, 0).reshape(BT, N, C)
    vx_in = vp_flat.reshape(C, B, T, N).transpose(1, 2, 3, 0).reshape(BT, N, C)

    thetas = []
    for sp in range(SPLITS):
        sl = slice(sp * BTs, (sp + 1) * BTs)
        top16 = sc_knn(cand3[sl], pts_for(xpos[sl]))
        thetas.append(top16.reshape(BTs * N, 16)[:, 15].reshape(BTs, 1, N))

    xr_parts = []
    for sp in range(SPLITS):
        sl = slice(sp * BTs, (sp + 1) * BTs)
        xr_parts.append(pl.pallas_call(
            _attn_kernel,
            grid=(BTs, N // R),
            in_specs=[
                pl.BlockSpec((1, R, 4), lambda bt, nt: (bt, nt, 0)),
                pl.BlockSpec((1, 3, J), lambda bt, nt: (bt, 0, 0)),
                pl.BlockSpec((1, 1, R), lambda bt, nt: (bt, 0, nt)),
                pl.BlockSpec((1, R, C), lambda bt, nt: (bt, nt, 0)),
                pl.BlockSpec((1, R, C), lambda bt, nt: (bt, nt, 0)),
                pl.BlockSpec((1, R, C), lambda bt, nt: (bt, nt, 0)),
                pl.BlockSpec((1, C, J), lambda bt, nt: (bt, 0, 0)),
                pl.BlockSpec((1, C, J), lambda bt, nt: (bt, 0, 0)),
            ],
            out_specs=pl.BlockSpec((1, R, C), lambda bt, nt: (bt, nt, 0)),
            out_shape=jax.ShapeDtypeStruct((BTs, N, C), jnp.float32),
        )(pos_pts[sl], cand3[sl], thetas[sp], q_in[sl], x_in[sl],
          vx_in[sl], kc[sl], vc[sl]))
    xr = jnp.concatenate(xr_parts, axis=0)

    xr_t = xr.reshape(B, T, N, C).transpose(1, 3, 0, 2).reshape(T, C, B * N)
    y = pl.pallas_call(
        _bn_mlp_kernel,
        grid=(T,),
        in_specs=[
            pl.BlockSpec((1, C, B * N), lambda t: (t, 0, 0)),
            pl.BlockSpec((1, C), lambda t: (0, 0)),
            pl.BlockSpec((1, C), lambda t: (0, 0)),
            pl.BlockSpec((Cmid, C), lambda t: (0, 0)),
            pl.BlockSpec((Cout, Cmid), lambda t: (0, 0)),
        ],
        out_specs=pl.BlockSpec((1, Cout, B * N), lambda t: (t, 0, 0)),
        out_shape=jax.ShapeDtypeStruct((T, Cout, B * N), jnp.float32),
    )(xr_t, gamma1.reshape(1, C), beta1.reshape(1, C), W1, W2)

    y = y.reshape(T, Cout, B, N).transpose(2, 1, 0, 3)
    return jnp.concatenate([x_all[:, :4], y], axis=1)
